# baseline (speedup 1.0000x reference)
import jax
import jax.numpy as jnp
import numpy as np
from jax.experimental import pallas as pl
from jax.experimental.pallas import tpu as pltpu

_BINS = 10
_N = 4 * 64 * 64 * 9 * 80
_COLS = 1024
_ROWS = _N // _COLS
_BR = 720
_GRID = _ROWS // _BR

_EDGES = np.arange(_BINS + 1, dtype=np.float32) / np.float32(_BINS)
_EDGES[_BINS] += np.float32(1e-6)


def _body(lbl_ref, x_ref, w_ref, out_ref, acc_ref):
    step = pl.program_id(0)

    @pl.when(step == 0)
    def _init():
        for i in range(_BINS):
            acc_ref[0, i] = 0.0
            acc_ref[1, i] = 0.0

    lbl = lbl_ref[...]
    x = x_ref[...]
    w = w_ref[...]

    p = jax.nn.sigmoid(x)
    g = jnp.where(lbl == 1, 1.0 - p, p)
    ce = (jnp.maximum(x, 0.0) - x * lbl.astype(jnp.float32)
          + jnp.log1p(jnp.exp(-jnp.abs(x))))
    valid = w > 0.0

    for i in range(_BINS):
        m = (g >= _EDGES[i]) & (g < _EDGES[i + 1]) & valid
        acc_ref[0, i] = acc_ref[0, i] + jnp.sum(m.astype(jnp.float32))
        acc_ref[1, i] = acc_ref[1, i] + jnp.sum(jnp.where(m, ce, 0.0))

    @pl.when(step == _GRID - 1)
    def _fin():
        tot = 0.0
        n = 0.0
        for i in range(_BINS):
            c = acc_ref[0, i]
            s = acc_ref[1, i]
            ne = c > 0.0
            tot += jnp.where(ne, s / jnp.maximum(c, 1.0), 0.0)
            n += jnp.where(ne, 1.0, 0.0)
        out_ref[0, 0] = jnp.where(n > 0.0, tot / jnp.maximum(n, 1.0), 0.0)


def kernel(class_labels, class_logits, label_weights):
    lbl = class_labels.reshape(_ROWS, _COLS)
    x = class_logits.reshape(_ROWS, _COLS)
    w = label_weights.reshape(_ROWS, _COLS)
    out = pl.pallas_call(
        _body,
        grid=(_GRID,),
        in_specs=[
            pl.BlockSpec((_BR, _COLS), lambda i: (i, 0)),
            pl.BlockSpec((_BR, _COLS), lambda i: (i, 0)),
            pl.BlockSpec((_BR, _COLS), lambda i: (i, 0)),
        ],
        out_specs=pl.BlockSpec(memory_space=pltpu.SMEM),
        out_shape=jax.ShapeDtypeStruct((1, 1), jnp.float32),
        scratch_shapes=[pltpu.SMEM((2, _BINS), jnp.float32)],
        compiler_params=pltpu.CompilerParams(
            dimension_semantics=("arbitrary",)),
    )(lbl, x, w)
    return out[0, 0]
